# Initial kernel scaffold; baseline (speedup 1.0000x reference)
import functools

import jax
import jax.numpy as jnp
from jax.experimental import pallas as pl
from jax.experimental.pallas import tpu as pltpu

_B, _S, _D = 4, 4096, 768
_DFF = 3072
_EPS = 1e-05
_K = _S // 2

_RB = 2048
_TB = 256


def _wvo_body(wv_ref, wo_ref, out_ref):
    out_ref[...] = jnp.dot(wv_ref[...], wo_ref[...],
                           preferred_element_type=jnp.float32)


def _router_body(x_ref, rw_ref, rb_ref, w_ref):
    logits = jnp.dot(x_ref[...], rw_ref[...],
                     preferred_element_type=jnp.float32)
    w_ref[...] = jax.nn.sigmoid(logits + rb_ref[0])


def _thr_body(w_ref, thr_ref):
    bits = jax.lax.bitcast_convert_type(w_ref[...], jnp.int32)

    def body(_, carry):
        lo, hi = carry
        mid = lo + (hi - lo) // 2
        cnt = jnp.sum((bits >= mid).astype(jnp.int32), axis=1, keepdims=True)
        ge = cnt >= _K
        return jnp.where(ge, mid, lo), jnp.where(ge, hi, mid)

    lo0 = jnp.zeros((_B, 1), jnp.int32)
    hi0 = jnp.full((_B, 1), 0x7F800000, jnp.int32)
    lo, _ = jax.lax.fori_loop(0, 31, body, (lo0, hi0))
    thr = jax.lax.bitcast_convert_type(lo, jnp.float32)
    thr_ref[...] = jnp.broadcast_to(thr, (_B, 128))


def _main_body(x_ref, w_ref, thr_ref, wvo_ref, g1_ref, g2_ref,
               wg_ref, wu_ref, wd_ref, out_ref):
    x = x_ref[0]
    n1 = x * jax.lax.rsqrt(jnp.mean(x * x, axis=-1, keepdims=True) + _EPS)
    n1 = n1 * g1_ref[...]
    attn = jnp.dot(n1, wvo_ref[...], preferred_element_type=jnp.float32)
    resid = x + attn
    n2 = resid * jax.lax.rsqrt(
        jnp.mean(resid * resid, axis=-1, keepdims=True) + _EPS)
    n2 = n2 * g2_ref[...]
    a = jnp.dot(n2, wg_ref[...], preferred_element_type=jnp.float32)
    b = jnp.dot(n2, wu_ref[...], preferred_element_type=jnp.float32)
    h = jax.nn.silu(a) * b
    ffn = jnp.dot(h, wd_ref[...], preferred_element_type=jnp.float32)
    osel = resid + ffn
    mask = w_ref[0] >= thr_ref[0:1, 0:1]
    out_ref[0] = jnp.where(mask, osel, x)


def kernel(hidden_states, router_w, router_b, wq, wk, wv, wo, g1, g2, wg, wu, wd):
    del wq, wk
    x = hidden_states

    wvo = pl.pallas_call(
        _wvo_body,
        out_shape=jax.ShapeDtypeStruct((_D, _D), jnp.float32),
    )(wv, wo)

    xf = x.reshape(_B * _S, _D)
    weights = pl.pallas_call(
        _router_body,
        grid=(_B * _S // _RB,),
        in_specs=[
            pl.BlockSpec((_RB, _D), lambda i: (i, 0)),
            pl.BlockSpec((_D, 1), lambda i: (0, 0)),
            pl.BlockSpec(memory_space=pltpu.SMEM),
        ],
        out_specs=pl.BlockSpec((_RB, 1), lambda i: (i, 0)),
        out_shape=jax.ShapeDtypeStruct((_B * _S, 1), jnp.float32),
    )(xf, router_w, router_b)

    thr = pl.pallas_call(
        _thr_body,
        out_shape=jax.ShapeDtypeStruct((_B, 128), jnp.float32),
    )(weights.reshape(_B, _S))

    wcol = weights.reshape(_B, _S, 1)
    out = pl.pallas_call(
        _main_body,
        grid=(_B, _S // _TB),
        in_specs=[
            pl.BlockSpec((1, _TB, _D), lambda b, s: (b, s, 0)),
            pl.BlockSpec((1, _TB, 1), lambda b, s: (b, s, 0)),
            pl.BlockSpec((1, 128), lambda b, s: (b, 0)),
            pl.BlockSpec((_D, _D), lambda b, s: (0, 0)),
            pl.BlockSpec((1, _D), lambda b, s: (0, 0)),
            pl.BlockSpec((1, _D), lambda b, s: (0, 0)),
            pl.BlockSpec((_D, _DFF), lambda b, s: (0, 0)),
            pl.BlockSpec((_D, _DFF), lambda b, s: (0, 0)),
            pl.BlockSpec((_DFF, _D), lambda b, s: (0, 0)),
        ],
        out_specs=pl.BlockSpec((1, _TB, _D), lambda b, s: (b, s, 0)),
        out_shape=jax.ShapeDtypeStruct((_B, _S, _D), jnp.float32),
    )(x, wcol, thr, wvo, g1.reshape(1, _D), g2.reshape(1, _D), wg, wu, wd)
    return out


# speedup vs baseline: 1.8114x; 1.8114x over previous
import functools

import jax
import jax.numpy as jnp
from jax.experimental import pallas as pl
from jax.experimental.pallas import tpu as pltpu

_B, _S, _D = 4, 4096, 768
_DFF = 3072
_EPS = 1e-05
_K = _S // 2

_RB = 2048
_TB = 256


def _wvo_body(wv_ref, wo_ref, out_ref):
    out_ref[...] = jnp.dot(wv_ref[...], wo_ref[...],
                           preferred_element_type=jnp.float32)


def _router_body(x_ref, rw_ref, rb_ref, w_ref):
    logits = jnp.dot(x_ref[...], rw_ref[...],
                     preferred_element_type=jnp.float32)
    w_ref[...] = jax.nn.sigmoid(logits + rb_ref[0])


def _thr_body(w_ref, thr_ref):
    bits = jax.lax.bitcast_convert_type(w_ref[...], jnp.int32)

    def body(_, carry):
        lo, hi = carry
        mid = lo + (hi - lo) // 2
        cnt = jnp.sum((bits >= mid).astype(jnp.int32), axis=1, keepdims=True)
        ge = cnt >= _K
        return jnp.where(ge, mid, lo), jnp.where(ge, hi, mid)

    lo0 = jnp.zeros((_B, 1), jnp.int32)
    hi0 = jnp.full((_B, 1), 0x7F800000, jnp.int32)
    lo, _ = jax.lax.fori_loop(0, 31, body, (lo0, hi0))
    thr = jax.lax.bitcast_convert_type(lo, jnp.float32)
    thr_ref[...] = jnp.broadcast_to(thr, (_B, 128))


def _main_body(x_ref, w_ref, thr_ref, wvo_ref, g1_ref, g2_ref,
               wg_ref, wu_ref, wd_ref, out_ref):
    x = x_ref[0]
    n1 = x * jax.lax.rsqrt(jnp.mean(x * x, axis=-1, keepdims=True) + _EPS)
    n1 = n1 * g1_ref[...]
    attn = jnp.dot(n1, wvo_ref[...], preferred_element_type=jnp.float32)
    resid = x + attn
    n2 = resid * jax.lax.rsqrt(
        jnp.mean(resid * resid, axis=-1, keepdims=True) + _EPS)
    n2 = n2 * g2_ref[...]
    a = jnp.dot(n2, wg_ref[...], preferred_element_type=jnp.float32)
    b = jnp.dot(n2, wu_ref[...], preferred_element_type=jnp.float32)
    h = jax.nn.silu(a) * b
    ffn = jnp.dot(h, wd_ref[...], preferred_element_type=jnp.float32)
    osel = resid + ffn
    mask = w_ref[0] >= thr_ref[0, 0:1, 0:1]
    out_ref[0] = jnp.where(mask, osel, x)


def kernel(hidden_states, router_w, router_b, wq, wk, wv, wo, g1, g2, wg, wu, wd):
    del wq, wk
    x = hidden_states

    wvo = pl.pallas_call(
        _wvo_body,
        out_shape=jax.ShapeDtypeStruct((_D, _D), jnp.float32),
    )(wv, wo)

    xf = x.reshape(_B * _S, _D)
    weights = pl.pallas_call(
        _router_body,
        grid=(_B * _S // _RB,),
        in_specs=[
            pl.BlockSpec((_RB, _D), lambda i: (i, 0)),
            pl.BlockSpec((_D, 1), lambda i: (0, 0)),
            pl.BlockSpec(memory_space=pltpu.SMEM),
        ],
        out_specs=pl.BlockSpec((_RB, 1), lambda i: (i, 0)),
        out_shape=jax.ShapeDtypeStruct((_B * _S, 1), jnp.float32),
    )(xf, router_w, router_b)

    thr = pl.pallas_call(
        _thr_body,
        out_shape=jax.ShapeDtypeStruct((_B, 128), jnp.float32),
    )(weights.reshape(_B, _S))

    wcol = weights.reshape(_B, _S, 1)
    out = pl.pallas_call(
        _main_body,
        grid=(_B, _S // _TB),
        in_specs=[
            pl.BlockSpec((1, _TB, _D), lambda b, s: (b, s, 0)),
            pl.BlockSpec((1, _TB, 1), lambda b, s: (b, s, 0)),
            pl.BlockSpec((1, 1, 128), lambda b, s: (b, 0, 0)),
            pl.BlockSpec((_D, _D), lambda b, s: (0, 0)),
            pl.BlockSpec((1, _D), lambda b, s: (0, 0)),
            pl.BlockSpec((1, _D), lambda b, s: (0, 0)),
            pl.BlockSpec((_D, _DFF), lambda b, s: (0, 0)),
            pl.BlockSpec((_D, _DFF), lambda b, s: (0, 0)),
            pl.BlockSpec((_DFF, _D), lambda b, s: (0, 0)),
        ],
        out_specs=pl.BlockSpec((1, _TB, _D), lambda b, s: (b, s, 0)),
        out_shape=jax.ShapeDtypeStruct((_B, _S, _D), jnp.float32),
    )(x, wcol, thr.reshape(_B, 1, 128), wvo,
      g1.reshape(1, _D), g2.reshape(1, _D), wg, wu, wd)
    return out
